# baseline (speedup 1.0000x reference)
import functools

import jax
import jax.numpy as jnp
from jax import lax
from jax.experimental import pallas as pl
from jax.experimental.pallas import tpu as pltpu
from jax.experimental.pallas import tpu_sc as plsc

B = 1024
F = 640
D = 256
NK = 64
KB = 8
PP_TILE = 128

SC_B = 256
TC_B = B - SC_B
_NC, _NS = 2, 16
_NW = _NC * _NS
_BPW = SC_B // _NW
_LANES = 16
_NCHUNK = F // _LANES


def _ppsum_body(pp_ref, out_ref):
    out_ref[...] = jnp.sum(pp_ref[...], axis=1).astype(jnp.float32)


def _sc_row_sum(slab_ref, row_ref):
    for g in range(_NCHUNK // 10):
        def body(s, carry):
            return tuple(
                c + slab_ref[s, pl.ds(_LANES * (g * 10 + j), _LANES)]
                for j, c in enumerate(carry)
            )
        init = tuple(jnp.zeros((_LANES,), jnp.int32) for _ in range(10))
        acc = lax.fori_loop(0, 64, body, init)
        for j in range(10):
            row_ref[pl.ds(_LANES * (g * 10 + j), _LANES)] = (
                acc[j].astype(jnp.float32))


def _sc_ppsum_kernel(pp_hbm, out_hbm, slab0, slab1, row_v, sem0, sem1):
    wid = lax.axis_index("s") * _NC + lax.axis_index("c")
    base = TC_B + wid * _BPW
    slabs = (slab0, slab1)
    sems = (sem0, sem1)
    cp0 = pltpu.make_async_copy(pp_hbm.at[base], slabs[0], sems[0])
    cp0.start()
    for i in range(_BPW):
        if i + 1 < _BPW:
            nxt = pltpu.make_async_copy(
                pp_hbm.at[base + i + 1], slabs[(i + 1) % 2], sems[(i + 1) % 2])
            nxt.start()
        pltpu.make_async_copy(
            pp_hbm.at[base + i], slabs[i % 2], sems[i % 2]).wait()
        _sc_row_sum(slabs[i % 2], row_v)
        pltpu.sync_copy(row_v, out_hbm.at[wid * _BPW + i])


def _sc_ppsum(piece_positions):
    mesh = plsc.VectorSubcoreMesh(core_axis_name="c", subcore_axis_name="s")
    run = functools.partial(
        pl.kernel,
        mesh=mesh,
        out_type=jax.ShapeDtypeStruct((SC_B, F), jnp.float32),
        scratch_types=[
            pltpu.VMEM((64, F), jnp.int32),
            pltpu.VMEM((64, F), jnp.int32),
            pltpu.VMEM((F,), jnp.float32),
            pltpu.SemaphoreType.DMA,
            pltpu.SemaphoreType.DMA,
        ],
    )(_sc_ppsum_kernel)
    return run(piece_positions)


def _main_body(ppsum_ref, w_ref, kings_ref, bias_ref, w1_ref, b1_ref,
               w2_ref, b2_ref, wout_ref, bout_ref, out_ref, xacc_ref):
    step = pl.program_id(0)
    kings = kings_ref[...]
    pp = ppsum_ref[...]

    acc = jnp.zeros((B, D), jnp.float32)
    for j in range(KB):
        k = step * KB + j
        wk = w_ref[j]
        m = (kings == k).astype(jnp.float32)
        msum = m[:, 0:1] + m[:, 1:2]
        z = jax.lax.dot_general(pp, wk[:F, :],
                                (((1,), (0,)), ((), ())),
                                preferred_element_type=jnp.float32)
        acc = acc + msum * (z + wk[F:F + 1, :])

    @pl.when(step == 0)
    def _init():
        xacc_ref[...] = acc

    @pl.when(step > 0)
    def _acc():
        xacc_ref[...] = xacc_ref[...] + acc

    @pl.when(step == NK // KB - 1)
    def _tail():
        x = xacc_ref[...] + bias_ref[...]
        x = jnp.clip(x, 0.0, 127.0)
        w1s = w1_ref[...][:, :D] + w1_ref[...][:, D:]
        h = jax.lax.dot_general(x, w1s, (((1,), (1,)), ((), ())),
                                preferred_element_type=jnp.float32)
        h = h + b1_ref[...]
        h = jnp.clip(jnp.floor(h * (1.0 / 64.0)), 0.0, 127.0)
        h = jax.lax.dot_general(h, w2_ref[...], (((1,), (1,)), ((), ())),
                                preferred_element_type=jnp.float32)
        h = h + b2_ref[...]
        h = jnp.clip(jnp.floor(h * (1.0 / 64.0)), 0.0, 127.0)
        v = jnp.sum(h * wout_ref[...]) + bout_ref[...]
        out_ref[...] = jnp.floor(v * (1.0 / 16.0))


def kernel(piece_positions, king_positions, input_weights, input_bias,
           w1, b1, w2, b2, w_out, b_out):
    ppsum_sc = _sc_ppsum(piece_positions)
    ppsum_tc = pl.pallas_call(
        _ppsum_body,
        grid=(TC_B // PP_TILE,),
        in_specs=[pl.BlockSpec((PP_TILE, 64, F), lambda i: (i, 0, 0))],
        out_specs=pl.BlockSpec((PP_TILE, F), lambda i: (i, 0)),
        out_shape=jax.ShapeDtypeStruct((TC_B, F), jnp.float32),
    )(piece_positions)
    ppsum = jnp.concatenate([ppsum_tc, ppsum_sc], axis=0)

    out = pl.pallas_call(
        _main_body,
        grid=(NK // KB,),
        in_specs=[
            pl.BlockSpec((B, F), lambda s: (0, 0)),
            pl.BlockSpec((KB, F + 1, D), lambda s: (s, 0, 0)),
            pl.BlockSpec((B, 2), lambda s: (0, 0)),
            pl.BlockSpec((1, D), lambda s: (0, 0)),
            pl.BlockSpec((32, 2 * D), lambda s: (0, 0)),
            pl.BlockSpec((1, 32), lambda s: (0, 0)),
            pl.BlockSpec((32, 32), lambda s: (0, 0)),
            pl.BlockSpec((1, 32), lambda s: (0, 0)),
            pl.BlockSpec((1, 32), lambda s: (0, 0)),
            pl.BlockSpec((1, 1), lambda s: (0, 0)),
        ],
        out_specs=pl.BlockSpec((1, 1), lambda s: (0, 0)),
        out_shape=jax.ShapeDtypeStruct((1, 1), jnp.float32),
        scratch_shapes=[pltpu.VMEM((B, D), jnp.float32)],
    )(
        ppsum,
        input_weights,
        king_positions,
        input_bias.reshape(1, D),
        w1,
        b1.reshape(1, 32),
        w2,
        b2.reshape(1, 32),
        w_out.reshape(1, 32),
        b_out.reshape(1, 1),
    )
    return out.reshape((1,))
